# Initial kernel scaffold; baseline (speedup 1.0000x reference)
import jax
import jax.numpy as jnp
from jax.experimental import pallas as pl


def kernel(ids, adj_0, adj_1, eid_0, eid_1, feats, W_prep, edge_emb_0, edge_emb_1, W_self, W_neigh, W_edge, Wa, va, W_fc, b_fc):
    raise NotImplementedError("write your pallas kernel here")


# speedup vs baseline: 2.2705x; 2.2705x over previous
import functools

import jax
import jax.numpy as jnp
from jax import lax
from jax.experimental import pallas as pl
from jax.experimental.pallas import tpu as pltpu
from jax.experimental.pallas import tpu_sc as plsc

N_NODES = 50000
D_FEAT = 128
D_EDGE = 16
N_EDGES = 500000
N_HEAD = 4
S = 10
B = 1024

NC = 2
NS = 16
NW = NC * NS

L1 = 2 * B * S
L2 = L1 * S
L1_PER_W = L1 // NW
L2_PER_W = L2 // NW


def _wid():
    return lax.axis_index("s") * NC + lax.axis_index("c")


@functools.cache
def _sc_mesh():
    return plsc.VectorSubcoreMesh(core_axis_name="c", subcore_axis_name="s")


COMB_W = 48


def _g1_body(ids_h, comb_h, out_h, idx_v, buf_v, sem):
    wid = _wid()
    base = wid * (B // NW)
    pltpu.sync_copy(ids_h.at[pl.ds(base, B // NW)], idx_v)
    pltpu.async_copy(comb_h.at[idx_v], buf_v, sem).wait()
    pltpu.sync_copy(buf_v, out_h.at[pl.ds(base, B // NW)])


@functools.cache
def _g1():
    return pl.kernel(
        _g1_body,
        out_type=[jax.ShapeDtypeStruct((B, COMB_W), jnp.int32)],
        mesh=_sc_mesh(),
        compiler_params=pltpu.CompilerParams(use_tc_tiling_on_sc=False),
        scratch_types=[
            pltpu.VMEM((B // NW,), jnp.int32),
            pltpu.VMEM((B // NW, COMB_W), jnp.int32),
            pltpu.SemaphoreType.DMA,
        ],
    )


_GCHUNK = 128


def _g2_body(cur1_h, e1f_h, ids_h, p_h, comb_h, ee0_h, ee1_h,
             x1_h, e1o_h, combo_h, x0_h,
             idx_v, eidx_v, xbuf_v, ebuf_v, cbuf_v,
             ids_v, x0buf_v, sem):
    wid = _wid()
    base = wid * L1_PER_W
    pltpu.sync_copy(cur1_h.at[pl.ds(base, L1_PER_W)], idx_v)
    pltpu.sync_copy(e1f_h.at[pl.ds(base, L1_PER_W)], eidx_v)

    nchunk = L1_PER_W // _GCHUNK

    for j in range(nchunk):
        sl = pl.ds(j * _GCHUNK, _GCHUNK)
        osl = pl.ds(base + j * _GCHUNK, _GCHUNK)
        pltpu.async_copy(p_h.at[idx_v.at[sl]], xbuf_v, sem).wait()
        pltpu.sync_copy(xbuf_v, x1_h.at[osl])
        pltpu.async_copy(comb_h.at[idx_v.at[sl]], cbuf_v, sem).wait()
        pltpu.sync_copy(cbuf_v, combo_h.at[osl])

    def do_edges(ee_h):
        for j in range(nchunk):
            sl = pl.ds(j * _GCHUNK, _GCHUNK)
            osl = pl.ds(base + j * _GCHUNK, _GCHUNK)
            pltpu.async_copy(ee_h.at[eidx_v.at[sl]], ebuf_v, sem).wait()
            pltpu.sync_copy(ebuf_v, e1o_h.at[osl])

    @pl.when(wid < NS)
    def _():
        do_edges(ee0_h)

    @pl.when(wid >= NS)
    def _():
        do_edges(ee1_h)

    xbase = wid * (B // NW)
    pltpu.sync_copy(ids_h.at[pl.ds(xbase, B // NW)], ids_v)
    pltpu.async_copy(p_h.at[ids_v], x0buf_v, sem).wait()
    pltpu.sync_copy(x0buf_v, x0_h.at[pl.ds(xbase, B // NW)])


@functools.cache
def _g2():
    return pl.kernel(
        _g2_body,
        out_type=[
            jax.ShapeDtypeStruct((L1, D_FEAT), jnp.float32),
            jax.ShapeDtypeStruct((L1, D_EDGE), jnp.float32),
            jax.ShapeDtypeStruct((L1, COMB_W), jnp.int32),
            jax.ShapeDtypeStruct((B, D_FEAT), jnp.float32),
        ],
        mesh=_sc_mesh(),
        compiler_params=pltpu.CompilerParams(use_tc_tiling_on_sc=False),
        scratch_types=[
            pltpu.VMEM((L1_PER_W,), jnp.int32),
            pltpu.VMEM((L1_PER_W,), jnp.int32),
            pltpu.VMEM((_GCHUNK, D_FEAT), jnp.float32),
            pltpu.VMEM((_GCHUNK, D_EDGE), jnp.float32),
            pltpu.VMEM((_GCHUNK, COMB_W), jnp.int32),
            pltpu.VMEM((B // NW,), jnp.int32),
            pltpu.VMEM((B // NW, D_FEAT), jnp.float32),
            pltpu.SemaphoreType.DMA,
        ],
    )


_CH = 32
_ROWS = _CH * S
_NCHUNK = L1_PER_W // _CH


def _g3_body(nbr2f_h, e2f_h, p_h, ee0_h, ee1_h, m1_h, me2_h,
             nidx_v, eidx_v, xbuf_v, ebuf_v, accx_v, acce_v, sem, sem2):
    wid = _wid()
    base_dst = wid * L1_PER_W
    base_src = wid * L2_PER_W
    pltpu.sync_copy(nbr2f_h.at[pl.ds(base_src, L2_PER_W)], nidx_v)
    pltpu.sync_copy(e2f_h.at[pl.ds(base_src, L2_PER_W)], eidx_v)

    def run(ee_h):
        def chunk_body(ch, carry):
            off = ch * _ROWS
            cs = []
            for (o, n) in ((0, 128), (128, 128), (256, 64)):
                cs.append(pltpu.async_copy(
                    p_h.at[nidx_v.at[pl.ds(off + o, n)]],
                    xbuf_v.at[pl.ds(o, n)], sem))
                cs.append(pltpu.async_copy(
                    ee_h.at[eidx_v.at[pl.ds(off + o, n)]],
                    ebuf_v.at[pl.ds(o, n)], sem2))
            for c in cs:
                c.wait()

            def dbody(d, carry2):
                r = d * S
                for c in range(D_FEAT // 16):
                    sl = pl.ds(c * 16, 16)
                    acc = xbuf_v[r, sl]
                    for s in range(1, S):
                        acc = acc + xbuf_v[r + s, sl]
                    accx_v[d, sl] = acc
                ae = ebuf_v[r, :]
                for s in range(1, S):
                    ae = ae + ebuf_v[r + s, :]
                acce_v[d, :] = ae
                return carry2

            lax.fori_loop(0, _CH, dbody, 0)
            pltpu.sync_copy(accx_v, m1_h.at[pl.ds(base_dst + ch * _CH, _CH)])
            pltpu.sync_copy(acce_v, me2_h.at[pl.ds(base_dst + ch * _CH, _CH)])
            return carry

        lax.fori_loop(0, _NCHUNK, chunk_body, 0)

    @pl.when(wid < NS)
    def _():
        run(ee0_h)

    @pl.when(wid >= NS)
    def _():
        run(ee1_h)


@functools.cache
def _g3():
    return pl.kernel(
        _g3_body,
        out_type=[
            jax.ShapeDtypeStruct((L1, D_FEAT), jnp.float32),
            jax.ShapeDtypeStruct((L1, D_EDGE), jnp.float32),
        ],
        mesh=_sc_mesh(),
        compiler_params=pltpu.CompilerParams(use_tc_tiling_on_sc=False),
        scratch_types=[
            pltpu.VMEM((L2_PER_W,), jnp.int32),
            pltpu.VMEM((L2_PER_W,), jnp.int32),
            pltpu.VMEM((_ROWS, D_FEAT), jnp.float32),
            pltpu.VMEM((_ROWS, D_EDGE), jnp.float32),
            pltpu.VMEM((_CH, D_FEAT), jnp.float32),
            pltpu.VMEM((_CH, D_EDGE), jnp.float32),
            pltpu.SemaphoreType.DMA,
            pltpu.SemaphoreType.DMA,
        ],
    )


def _prep_body(f_ref, w_ref, o_ref):
    o_ref[...] = jnp.dot(f_ref[...], w_ref[...],
                         preferred_element_type=jnp.float32)


def _project_table(feats, W_prep):
    blk = 1000
    return pl.pallas_call(
        _prep_body,
        grid=(N_NODES // blk,),
        in_specs=[
            pl.BlockSpec((blk, D_FEAT), lambda i: (i, 0)),
            pl.BlockSpec((D_FEAT, D_FEAT), lambda i: (0, 0)),
        ],
        out_specs=pl.BlockSpec((blk, D_FEAT), lambda i: (i, 0)),
        out_shape=jax.ShapeDtypeStruct((N_NODES, D_FEAT), jnp.float32),
    )(feats, W_prep)


def _relu(x):
    return jnp.maximum(x, 0.0)


def _dense_body(x0, x1r, e1r, m1r, me2r,
                w0a, w0b, w0c, w1a, w1b, w1c, wea, web, wec,
                wa, va, wfc, bfc, logits_o, w_o):
    X0 = x0[...]
    outs = []
    for mp in range(2):
        X1r = x1r[mp]
        E1r = e1r[mp]
        M1r = m1r[mp]
        ME2r = me2r[mp]
        mx = X1r[:, 0:D_FEAT]
        me = E1r[:, 0:D_EDGE]
        for s in range(1, S):
            mx = mx + X1r[:, s * D_FEAT:(s + 1) * D_FEAT]
            me = me + E1r[:, s * D_EDGE:(s + 1) * D_EDGE]
        mx = mx * (1.0 / S)
        me = me * (1.0 / S)
        dot = lambda a, b: jnp.dot(a, b, preferred_element_type=jnp.float32)
        new0 = _relu(dot(X0, w0a[mp]) + dot(mx, w0b[mp]) + dot(me, w0c[mp]))
        a0 = dot(new0, wea[mp])
        acc1 = jnp.zeros((B, D_FEAT), jnp.float32)
        accE = jnp.zeros((B, D_EDGE), jnp.float32)
        for s in range(S):
            xs = X1r[:, s * D_FEAT:(s + 1) * D_FEAT]
            ms = M1r[:, s * D_FEAT:(s + 1) * D_FEAT]
            es = E1r[:, s * D_EDGE:(s + 1) * D_EDGE]
            mes = ME2r[:, s * D_EDGE:(s + 1) * D_EDGE]
            n1 = _relu(dot(xs, w0a[mp]) + dot(ms * (1.0 / S), w0b[mp])
                       + dot(mes * (1.0 / S), w0c[mp]))
            acc1 = acc1 + n1
            accE = accE + jnp.tanh(a0 + dot(n1, web[mp]) + dot(es, wec[mp]))
        out = _relu(dot(new0, w1a[mp]) + dot(acc1 * (1.0 / S), w1b[mp])
                    + dot(accE * (1.0 / S), w1c[mp]))
        outs.append(out)

    Wa = wa[...]
    t0 = jnp.tanh(jnp.dot(outs[0], Wa, preferred_element_type=jnp.float32))
    t1 = jnp.tanh(jnp.dot(outs[1], Wa, preferred_element_type=jnp.float32))
    dn = (((1,), (1,)), ((), ()))
    a0c = lax.dot_general(t0, va[...], dn,
                          preferred_element_type=jnp.float32)
    a1c = lax.dot_general(t1, va[...], dn,
                          preferred_element_type=jnp.float32)
    m = jnp.maximum(a0c, a1c)
    e0 = jnp.exp(a0c - m)
    e1 = jnp.exp(a1c - m)
    den = e0 + e1
    w0 = e0 / den
    w1 = e1 / den
    o = w0 * outs[0] + w1 * outs[1]
    nrm = jnp.sqrt(jnp.sum(o * o, axis=1, keepdims=True))
    o = o / (nrm + 1e-12)
    logits_o[...] = (jnp.dot(o, wfc[...], preferred_element_type=jnp.float32)
                     + bfc[...])
    a0r = lax.dot_general(va[...], t0, dn,
                          preferred_element_type=jnp.float32)
    a1r = lax.dot_general(va[...], t1, dn,
                          preferred_element_type=jnp.float32)
    mr = jnp.maximum(a0r, a1r)
    er0 = jnp.exp(a0r - mr)
    er1 = jnp.exp(a1r - mr)
    denr = er0 + er1
    w_o[0:1, :] = er0 / denr
    w_o[1:2, :] = er1 / denr


def _dense_stage(X0, X1r, E1r, M1r, ME2r, weights):
    (w0a, w0b, w0c, w1a, w1b, w1c, wea, web, wec, wa, va, wfc, bfc) = weights
    return pl.pallas_call(
        _dense_body,
        out_shape=[
            jax.ShapeDtypeStruct((B, 8), jnp.float32),
            jax.ShapeDtypeStruct((2, B), jnp.float32),
        ],
    )(X0, X1r, E1r, M1r, ME2r,
      w0a, w0b, w0c, w1a, w1b, w1c, wea, web, wec, wa, va, wfc, bfc)


def _build_head_weights(W_self_i, W_neigh_i):
    za = jnp.zeros((D_FEAT, 16), jnp.float32)
    zc = jnp.zeros((D_EDGE, 16), jnp.float32)
    wa = jnp.concatenate(
        [jnp.concatenate([W_self_i[h], za], axis=1) for h in range(N_HEAD)],
        axis=1)
    wb = jnp.concatenate(
        [jnp.concatenate([za, W_neigh_i[h][:D_FEAT]], axis=1)
         for h in range(N_HEAD)], axis=1)
    wc = jnp.concatenate(
        [jnp.concatenate([zc, W_neigh_i[h][D_FEAT:]], axis=1)
         for h in range(N_HEAD)], axis=1)
    return wa, wb, wc


def kernel(ids, adj_0, adj_1, eid_0, eid_1, feats, W_prep,
           edge_emb_0, edge_emb_1, W_self, W_neigh, W_edge,
           Wa, va, W_fc, b_fc):
    ids = ids.astype(jnp.int32)
    adj_0 = adj_0.astype(jnp.int32)
    adj_1 = adj_1.astype(jnp.int32)
    eid_0 = eid_0.astype(jnp.int32)
    eid_1 = eid_1.astype(jnp.int32)

    P = _project_table(feats, W_prep)

    comb = jnp.concatenate(
        [adj_0, eid_0, adj_1, eid_1,
         jnp.zeros((N_NODES, COMB_W - 4 * S), jnp.int32)], axis=1)

    (co0,) = _g1()(ids, comb)
    cur1f = jnp.concatenate([co0[:, 0:S].reshape(-1),
                             co0[:, 2 * S:3 * S].reshape(-1)])
    e1f = jnp.concatenate([co0[:, S:2 * S].reshape(-1),
                           co0[:, 3 * S:4 * S].reshape(-1)])

    X1, E1, combL1, X0 = _g2()(cur1f, e1f, ids, P, comb,
                               edge_emb_0, edge_emb_1)

    half = L1 // 2
    nbr2f = jnp.concatenate([combL1[:half, 0:S].reshape(-1),
                             combL1[half:, 2 * S:3 * S].reshape(-1)])
    e2f = jnp.concatenate([combL1[:half, S:2 * S].reshape(-1),
                           combL1[half:, 3 * S:4 * S].reshape(-1)])

    M1sum, ME2sum = _g3()(nbr2f, e2f, P, edge_emb_0, edge_emb_1)

    X1r = X1.reshape(2, B, S * D_FEAT)
    E1r = E1.reshape(2, B, S * D_EDGE)
    M1r = M1sum.reshape(2, B, S * D_FEAT)
    ME2r = ME2sum.reshape(2, B, S * D_EDGE)

    w0 = [_build_head_weights(W_self[mp, 0], W_neigh[mp, 0]) for mp in range(2)]
    w1 = [_build_head_weights(W_self[mp, 1], W_neigh[mp, 1]) for mp in range(2)]
    stack = lambda xs: jnp.stack(xs, axis=0)
    w0a, w0b, w0c = (stack([w0[mp][i] for mp in range(2)]) for i in range(3))
    w1a, w1b, w1c = (stack([w1[mp][i] for mp in range(2)]) for i in range(3))
    wea = W_edge[:, :D_FEAT, :]
    web = W_edge[:, D_FEAT:2 * D_FEAT, :]
    wec = W_edge[:, 2 * D_FEAT:, :]

    logits, w = _dense_stage(
        X0, X1r, E1r, M1r, ME2r,
        (w0a, w0b, w0c, w1a, w1b, w1c, wea, web, wec,
         Wa, va.reshape(1, -1), W_fc, b_fc.reshape(1, -1)))
    return (logits, w)


# speedup vs baseline: 2.4108x; 1.0618x over previous
import functools

import jax
import jax.numpy as jnp
from jax import lax
from jax.experimental import pallas as pl
from jax.experimental.pallas import tpu as pltpu
from jax.experimental.pallas import tpu_sc as plsc

N_NODES = 50000
D_FEAT = 128
D_EDGE = 16
N_EDGES = 500000
N_HEAD = 4
S = 10
B = 1024

NC = 2
NS = 16
NW = NC * NS

L1 = 2 * B * S
L2 = L1 * S
L1_PER_W = L1 // NW
L2_PER_W = L2 // NW


def _wid():
    return lax.axis_index("s") * NC + lax.axis_index("c")


@functools.cache
def _sc_mesh():
    return plsc.VectorSubcoreMesh(core_axis_name="c", subcore_axis_name="s")


COMB_W = 48


def _g1_body(ids_h, comb_h, out_h, idx_v, buf_v, sem):
    wid = _wid()
    base = wid * (B // NW)
    pltpu.sync_copy(ids_h.at[pl.ds(base, B // NW)], idx_v)
    pltpu.async_copy(comb_h.at[idx_v], buf_v, sem).wait()
    pltpu.sync_copy(buf_v, out_h.at[pl.ds(base, B // NW)])


@functools.cache
def _g1():
    return pl.kernel(
        _g1_body,
        out_type=[jax.ShapeDtypeStruct((B, COMB_W), jnp.int32)],
        mesh=_sc_mesh(),
        compiler_params=pltpu.CompilerParams(use_tc_tiling_on_sc=False),
        scratch_types=[
            pltpu.VMEM((B // NW,), jnp.int32),
            pltpu.VMEM((B // NW, COMB_W), jnp.int32),
            pltpu.SemaphoreType.DMA,
        ],
    )


_GCHUNK = 128


def _g2_body(cur1_h, e1f_h, ids_h, p_h, comb_h, ee0_h, ee1_h,
             x1_h, e1o_h, combo_h, x0_h,
             idx_v, eidx_v, xbuf_v, ebuf_v, cbuf_v,
             ids_v, x0buf_v, sem):
    wid = _wid()
    base = wid * L1_PER_W
    pltpu.sync_copy(cur1_h.at[pl.ds(base, L1_PER_W)], idx_v)
    pltpu.sync_copy(e1f_h.at[pl.ds(base, L1_PER_W)], eidx_v)

    nchunk = L1_PER_W // _GCHUNK

    for j in range(nchunk):
        sl = pl.ds(j * _GCHUNK, _GCHUNK)
        osl = pl.ds(base + j * _GCHUNK, _GCHUNK)
        pltpu.async_copy(p_h.at[idx_v.at[sl]], xbuf_v, sem).wait()
        pltpu.sync_copy(xbuf_v, x1_h.at[osl])
        pltpu.async_copy(comb_h.at[idx_v.at[sl]], cbuf_v, sem).wait()
        pltpu.sync_copy(cbuf_v, combo_h.at[osl])

    def do_edges(ee_h):
        for j in range(nchunk):
            sl = pl.ds(j * _GCHUNK, _GCHUNK)
            osl = pl.ds(base + j * _GCHUNK, _GCHUNK)
            pltpu.async_copy(ee_h.at[eidx_v.at[sl]], ebuf_v, sem).wait()
            pltpu.sync_copy(ebuf_v, e1o_h.at[osl])

    @pl.when(wid < NS)
    def _():
        do_edges(ee0_h)

    @pl.when(wid >= NS)
    def _():
        do_edges(ee1_h)

    xbase = wid * (B // NW)
    pltpu.sync_copy(ids_h.at[pl.ds(xbase, B // NW)], ids_v)
    pltpu.async_copy(p_h.at[ids_v], x0buf_v, sem).wait()
    pltpu.sync_copy(x0buf_v, x0_h.at[pl.ds(xbase, B // NW)])


@functools.cache
def _g2():
    return pl.kernel(
        _g2_body,
        out_type=[
            jax.ShapeDtypeStruct((L1, D_FEAT), jnp.float32),
            jax.ShapeDtypeStruct((L1, D_EDGE), jnp.float32),
            jax.ShapeDtypeStruct((L1, COMB_W), jnp.int32),
            jax.ShapeDtypeStruct((B, D_FEAT), jnp.float32),
        ],
        mesh=_sc_mesh(),
        compiler_params=pltpu.CompilerParams(use_tc_tiling_on_sc=False),
        scratch_types=[
            pltpu.VMEM((L1_PER_W,), jnp.int32),
            pltpu.VMEM((L1_PER_W,), jnp.int32),
            pltpu.VMEM((_GCHUNK, D_FEAT), jnp.float32),
            pltpu.VMEM((_GCHUNK, D_EDGE), jnp.float32),
            pltpu.VMEM((_GCHUNK, COMB_W), jnp.int32),
            pltpu.VMEM((B // NW,), jnp.int32),
            pltpu.VMEM((B // NW, D_FEAT), jnp.float32),
            pltpu.SemaphoreType.DMA,
        ],
    )


_CH = 20
_ROWS = _CH * S
_NCHUNK = L1_PER_W // _CH
_SPLITS = ((0, 128), (128, _ROWS - 128))


def _g3_body(nbr2f_h, e2f_h, p_h, ee0_h, ee1_h, m1_h, me2_h,
             nidx_v, eidx_v, xbuf_v, ebuf_v, accx_v, acce_v,
             semx0, semx1, seme0, seme1):
    wid = _wid()
    base_dst = wid * L1_PER_W
    base_src = wid * L2_PER_W
    pltpu.sync_copy(nbr2f_h.at[pl.ds(base_src, L2_PER_W)], nidx_v)
    pltpu.sync_copy(e2f_h.at[pl.ds(base_src, L2_PER_W)], eidx_v)

    semx = (semx0, semx1)
    seme = (seme0, seme1)

    def run(ee_h):
        def fire(ch, b):
            off = ch * _ROWS
            for (o, n) in _SPLITS:
                pltpu.async_copy(p_h.at[nidx_v.at[pl.ds(off + o, n)]],
                                 xbuf_v.at[b].at[pl.ds(o, n)], semx[b])
                pltpu.async_copy(ee_h.at[eidx_v.at[pl.ds(off + o, n)]],
                                 ebuf_v.at[b].at[pl.ds(o, n)], seme[b])

        def process(ch, b):
            for (o, n) in _SPLITS:
                pltpu.make_async_copy(p_h.at[pl.ds(0, n)],
                                      xbuf_v.at[b].at[pl.ds(o, n)],
                                      semx[b]).wait()
                pltpu.make_async_copy(ee_h.at[pl.ds(0, n)],
                                      ebuf_v.at[b].at[pl.ds(o, n)],
                                      seme[b]).wait()
            xb = xbuf_v.at[b]
            eb = ebuf_v.at[b]

            def dbody(d, carry2):
                r = d * S
                for c in range(D_FEAT // 16):
                    sl = pl.ds(c * 16, 16)
                    acc = xb[r, sl]
                    for s in range(1, S):
                        acc = acc + xb[r + s, sl]
                    accx_v[d, sl] = acc
                ae = eb[r, :]
                for s in range(1, S):
                    ae = ae + eb[r + s, :]
                acce_v[d, :] = ae
                return carry2

            lax.fori_loop(0, _CH, dbody, 0)
            pltpu.sync_copy(accx_v, m1_h.at[pl.ds(base_dst + ch * _CH, _CH)])
            pltpu.sync_copy(acce_v, me2_h.at[pl.ds(base_dst + ch * _CH, _CH)])

        fire(0, 0)

        def pair_body(g, carry):
            ch0 = g * 2

            @pl.when(ch0 + 1 < _NCHUNK)
            def _():
                fire(ch0 + 1, 1)

            process(ch0, 0)

            @pl.when(ch0 + 1 < _NCHUNK)
            def _():
                @pl.when(ch0 + 2 < _NCHUNK)
                def _():
                    fire(ch0 + 2, 0)

                process(ch0 + 1, 1)

            return carry

        lax.fori_loop(0, (_NCHUNK + 1) // 2, pair_body, 0)

    @pl.when(wid < NS)
    def _():
        run(ee0_h)

    @pl.when(wid >= NS)
    def _():
        run(ee1_h)


@functools.cache
def _g3():
    return pl.kernel(
        _g3_body,
        out_type=[
            jax.ShapeDtypeStruct((L1, D_FEAT), jnp.float32),
            jax.ShapeDtypeStruct((L1, D_EDGE), jnp.float32),
        ],
        mesh=_sc_mesh(),
        compiler_params=pltpu.CompilerParams(use_tc_tiling_on_sc=False),
        scratch_types=[
            pltpu.VMEM((L2_PER_W,), jnp.int32),
            pltpu.VMEM((L2_PER_W,), jnp.int32),
            pltpu.VMEM((2, _ROWS, D_FEAT), jnp.float32),
            pltpu.VMEM((2, _ROWS, D_EDGE), jnp.float32),
            pltpu.VMEM((_CH, D_FEAT), jnp.float32),
            pltpu.VMEM((_CH, D_EDGE), jnp.float32),
            pltpu.SemaphoreType.DMA,
            pltpu.SemaphoreType.DMA,
            pltpu.SemaphoreType.DMA,
            pltpu.SemaphoreType.DMA,
        ],
    )


def _prep_body(f_ref, w_ref, o_ref):
    o_ref[...] = jnp.dot(f_ref[...], w_ref[...],
                         preferred_element_type=jnp.float32)


def _project_table(feats, W_prep):
    blk = 1000
    return pl.pallas_call(
        _prep_body,
        grid=(N_NODES // blk,),
        in_specs=[
            pl.BlockSpec((blk, D_FEAT), lambda i: (i, 0)),
            pl.BlockSpec((D_FEAT, D_FEAT), lambda i: (0, 0)),
        ],
        out_specs=pl.BlockSpec((blk, D_FEAT), lambda i: (i, 0)),
        out_shape=jax.ShapeDtypeStruct((N_NODES, D_FEAT), jnp.float32),
    )(feats, W_prep)


def _relu(x):
    return jnp.maximum(x, 0.0)


def _dense_body(x0, x1r, e1r, m1r, me2r,
                w0a, w0b, w0c, w1a, w1b, w1c, wea, web, wec,
                wa, va, wfc, bfc, logits_o, w_o):
    X0 = x0[...]
    outs = []
    for mp in range(2):
        X1r = x1r[mp]
        E1r = e1r[mp]
        M1r = m1r[mp]
        ME2r = me2r[mp]
        mx = X1r[:, 0:D_FEAT]
        me = E1r[:, 0:D_EDGE]
        for s in range(1, S):
            mx = mx + X1r[:, s * D_FEAT:(s + 1) * D_FEAT]
            me = me + E1r[:, s * D_EDGE:(s + 1) * D_EDGE]
        mx = mx * (1.0 / S)
        me = me * (1.0 / S)
        dot = lambda a, b: jnp.dot(a, b, preferred_element_type=jnp.float32)
        new0 = _relu(dot(X0, w0a[mp]) + dot(mx, w0b[mp]) + dot(me, w0c[mp]))
        a0 = dot(new0, wea[mp])
        acc1 = jnp.zeros((B, D_FEAT), jnp.float32)
        accE = jnp.zeros((B, D_EDGE), jnp.float32)
        for s in range(S):
            xs = X1r[:, s * D_FEAT:(s + 1) * D_FEAT]
            ms = M1r[:, s * D_FEAT:(s + 1) * D_FEAT]
            es = E1r[:, s * D_EDGE:(s + 1) * D_EDGE]
            mes = ME2r[:, s * D_EDGE:(s + 1) * D_EDGE]
            n1 = _relu(dot(xs, w0a[mp]) + dot(ms * (1.0 / S), w0b[mp])
                       + dot(mes * (1.0 / S), w0c[mp]))
            acc1 = acc1 + n1
            accE = accE + jnp.tanh(a0 + dot(n1, web[mp]) + dot(es, wec[mp]))
        out = _relu(dot(new0, w1a[mp]) + dot(acc1 * (1.0 / S), w1b[mp])
                    + dot(accE * (1.0 / S), w1c[mp]))
        outs.append(out)

    Wa = wa[...]
    t0 = jnp.tanh(jnp.dot(outs[0], Wa, preferred_element_type=jnp.float32))
    t1 = jnp.tanh(jnp.dot(outs[1], Wa, preferred_element_type=jnp.float32))
    dn = (((1,), (1,)), ((), ()))
    a0c = lax.dot_general(t0, va[...], dn,
                          preferred_element_type=jnp.float32)
    a1c = lax.dot_general(t1, va[...], dn,
                          preferred_element_type=jnp.float32)
    m = jnp.maximum(a0c, a1c)
    e0 = jnp.exp(a0c - m)
    e1 = jnp.exp(a1c - m)
    den = e0 + e1
    w0 = e0 / den
    w1 = e1 / den
    o = w0 * outs[0] + w1 * outs[1]
    nrm = jnp.sqrt(jnp.sum(o * o, axis=1, keepdims=True))
    o = o / (nrm + 1e-12)
    logits_o[...] = (jnp.dot(o, wfc[...], preferred_element_type=jnp.float32)
                     + bfc[...])
    a0r = lax.dot_general(va[...], t0, dn,
                          preferred_element_type=jnp.float32)
    a1r = lax.dot_general(va[...], t1, dn,
                          preferred_element_type=jnp.float32)
    mr = jnp.maximum(a0r, a1r)
    er0 = jnp.exp(a0r - mr)
    er1 = jnp.exp(a1r - mr)
    denr = er0 + er1
    w_o[0:1, :] = er0 / denr
    w_o[1:2, :] = er1 / denr


def _dense_stage(X0, X1r, E1r, M1r, ME2r, weights):
    (w0a, w0b, w0c, w1a, w1b, w1c, wea, web, wec, wa, va, wfc, bfc) = weights
    return pl.pallas_call(
        _dense_body,
        out_shape=[
            jax.ShapeDtypeStruct((B, 8), jnp.float32),
            jax.ShapeDtypeStruct((2, B), jnp.float32),
        ],
    )(X0, X1r, E1r, M1r, ME2r,
      w0a, w0b, w0c, w1a, w1b, w1c, wea, web, wec, wa, va, wfc, bfc)


def _build_head_weights(W_self_i, W_neigh_i):
    za = jnp.zeros((D_FEAT, 16), jnp.float32)
    zc = jnp.zeros((D_EDGE, 16), jnp.float32)
    wa = jnp.concatenate(
        [jnp.concatenate([W_self_i[h], za], axis=1) for h in range(N_HEAD)],
        axis=1)
    wb = jnp.concatenate(
        [jnp.concatenate([za, W_neigh_i[h][:D_FEAT]], axis=1)
         for h in range(N_HEAD)], axis=1)
    wc = jnp.concatenate(
        [jnp.concatenate([zc, W_neigh_i[h][D_FEAT:]], axis=1)
         for h in range(N_HEAD)], axis=1)
    return wa, wb, wc


def kernel(ids, adj_0, adj_1, eid_0, eid_1, feats, W_prep,
           edge_emb_0, edge_emb_1, W_self, W_neigh, W_edge,
           Wa, va, W_fc, b_fc):
    ids = ids.astype(jnp.int32)
    adj_0 = adj_0.astype(jnp.int32)
    adj_1 = adj_1.astype(jnp.int32)
    eid_0 = eid_0.astype(jnp.int32)
    eid_1 = eid_1.astype(jnp.int32)

    P = _project_table(feats, W_prep)

    comb = jnp.concatenate(
        [adj_0, eid_0, adj_1, eid_1,
         jnp.zeros((N_NODES, COMB_W - 4 * S), jnp.int32)], axis=1)

    (co0,) = _g1()(ids, comb)
    cur1f = jnp.concatenate([co0[:, 0:S].reshape(-1),
                             co0[:, 2 * S:3 * S].reshape(-1)])
    e1f = jnp.concatenate([co0[:, S:2 * S].reshape(-1),
                           co0[:, 3 * S:4 * S].reshape(-1)])

    X1, E1, combL1, X0 = _g2()(cur1f, e1f, ids, P, comb,
                               edge_emb_0, edge_emb_1)

    half = L1 // 2
    nbr2f = jnp.concatenate([combL1[:half, 0:S].reshape(-1),
                             combL1[half:, 2 * S:3 * S].reshape(-1)])
    e2f = jnp.concatenate([combL1[:half, S:2 * S].reshape(-1),
                           combL1[half:, 3 * S:4 * S].reshape(-1)])

    M1sum, ME2sum = _g3()(nbr2f, e2f, P, edge_emb_0, edge_emb_1)

    X1r = X1.reshape(2, B, S * D_FEAT)
    E1r = E1.reshape(2, B, S * D_EDGE)
    M1r = M1sum.reshape(2, B, S * D_FEAT)
    ME2r = ME2sum.reshape(2, B, S * D_EDGE)

    w0 = [_build_head_weights(W_self[mp, 0], W_neigh[mp, 0]) for mp in range(2)]
    w1 = [_build_head_weights(W_self[mp, 1], W_neigh[mp, 1]) for mp in range(2)]
    stack = lambda xs: jnp.stack(xs, axis=0)
    w0a, w0b, w0c = (stack([w0[mp][i] for mp in range(2)]) for i in range(3))
    w1a, w1b, w1c = (stack([w1[mp][i] for mp in range(2)]) for i in range(3))
    wea = W_edge[:, :D_FEAT, :]
    web = W_edge[:, D_FEAT:2 * D_FEAT, :]
    wec = W_edge[:, 2 * D_FEAT:, :]

    logits, w = _dense_stage(
        X0, X1r, E1r, M1r, ME2r,
        (w0a, w0b, w0c, w1a, w1b, w1c, wea, web, wec,
         Wa, va.reshape(1, -1), W_fc, b_fc.reshape(1, -1)))
    return (logits, w)
